# Initial kernel scaffold; baseline (speedup 1.0000x reference)
import jax
import jax.numpy as jnp
from jax.experimental import pallas as pl


def kernel(input_coords, tables):
    raise NotImplementedError("write your pallas kernel here")


# speedup vs baseline: 11.1944x; 11.1944x over previous
import functools

import jax
import jax.numpy as jnp
from jax import lax
from jax.experimental import pallas as pl
from jax.experimental.pallas import tpu as pltpu
from jax.experimental.pallas import tpu_sc as plsc

N_LEVELS = 16
N_MIN_F = 16.0
N_MAX_F = 512.0
HASH_EXP = 19
T = 2 ** HASH_EXP
MASK = T - 1
PI2_I32 = -1640531535
NW = 32


def _make_kernel(n_coords, chunk):
    per_w = n_coords // NW
    n_chunks = per_w // chunk
    assert per_w % chunk == 0 and chunk % 16 == 0

    mesh = plsc.VectorSubcoreMesh(core_axis_name="c", subcore_axis_name="s")

    @functools.partial(
        pl.kernel,
        out_type=jax.ShapeDtypeStruct((n_coords * 32,), jnp.float32),
        mesh=mesh,
        scratch_types=dict(
            res_v=pltpu.VMEM((N_LEVELS,), jnp.int32),
            xs_v=pltpu.VMEM((chunk,), jnp.int32),
            ys_v=pltpu.VMEM((chunk,), jnp.int32),
            i00a=pltpu.VMEM((chunk,), jnp.int32),
            i00b=pltpu.VMEM((chunk,), jnp.int32),
            i01a=pltpu.VMEM((chunk,), jnp.int32),
            i01b=pltpu.VMEM((chunk,), jnp.int32),
            i10a=pltpu.VMEM((chunk,), jnp.int32),
            i10b=pltpu.VMEM((chunk,), jnp.int32),
            i11a=pltpu.VMEM((chunk,), jnp.int32),
            i11b=pltpu.VMEM((chunk,), jnp.int32),
            wx_v=pltpu.VMEM((chunk,), jnp.float32),
            wy_v=pltpu.VMEM((chunk,), jnp.float32),
            f00a=pltpu.VMEM((chunk,), jnp.float32),
            f00b=pltpu.VMEM((chunk,), jnp.float32),
            f01a=pltpu.VMEM((chunk,), jnp.float32),
            f01b=pltpu.VMEM((chunk,), jnp.float32),
            f10a=pltpu.VMEM((chunk,), jnp.float32),
            f10b=pltpu.VMEM((chunk,), jnp.float32),
            f11a=pltpu.VMEM((chunk,), jnp.float32),
            f11b=pltpu.VMEM((chunk,), jnp.float32),
            out_v=pltpu.VMEM((chunk * 32,), jnp.float32),
            sem=pltpu.SemaphoreType.DMA,
        ),
        compiler_params=pltpu.CompilerParams(needs_layout_passes=False),
    )
    def ngp_kernel(tab, xs, ys, res, out, res_v, xs_v, ys_v,
                   i00a, i00b, i01a, i01b, i10a, i10b, i11a, i11b,
                   wx_v, wy_v,
                   f00a, f00b, f01a, f01b, f10a, f10b, f11a, f11b,
                   out_v, sem):
        pltpu.sync_copy(res, res_v)
        wid = lax.axis_index("s") * 2 + lax.axis_index("c")
        iota = lax.iota(jnp.int32, 16)

        def chunk_body(ci, carry0):
            base = pl.multiple_of(wid * per_w + ci * chunk, chunk)
            pltpu.sync_copy(xs.at[pl.ds(base, chunk)], xs_v)
            pltpu.sync_copy(ys.at[pl.ds(base, chunk)], ys_v)

            def level_body(l, carry1):
                r = plsc.load_gather(res_v, [jnp.full((16,), l, jnp.int32)])
                off2 = l * (T * 2)

                def idx_body(j, carry, r=r, off2=off2):
                    s = j * 16
                    cx = xs_v[pl.ds(s, 16)]
                    cy = ys_v[pl.ds(s, 16)]
                    px = cx * r
                    py = cy * r
                    x0 = px >> 9
                    y0 = py >> 9
                    hy0 = y0 * PI2_I32
                    hy1 = hy0 + PI2_I32
                    x1 = x0 + 1
                    v00 = (((x0 ^ hy0) & MASK) << 1) + off2
                    v01 = (((x0 ^ hy1) & MASK) << 1) + off2
                    v10 = (((x1 ^ hy0) & MASK) << 1) + off2
                    v11 = (((x1 ^ hy1) & MASK) << 1) + off2
                    i00a[pl.ds(s, 16)] = v00
                    i00b[pl.ds(s, 16)] = v00 + 1
                    i01a[pl.ds(s, 16)] = v01
                    i01b[pl.ds(s, 16)] = v01 + 1
                    i10a[pl.ds(s, 16)] = v10
                    i10b[pl.ds(s, 16)] = v10 + 1
                    i11a[pl.ds(s, 16)] = v11
                    i11b[pl.ds(s, 16)] = v11 + 1
                    wx_v[pl.ds(s, 16)] = (px & 511).astype(jnp.float32) * (1.0 / 512.0)
                    wy_v[pl.ds(s, 16)] = (py & 511).astype(jnp.float32) * (1.0 / 512.0)
                    return carry

                lax.fori_loop(jnp.int32(0), jnp.int32(chunk // 16), idx_body,
                              jnp.int32(0))

                cps = [pltpu.async_copy(tab.at[i], f, sem)
                       for i, f in ((i00a, f00a), (i00b, f00b),
                                    (i01a, f01a), (i01b, f01b),
                                    (i10a, f10a), (i10b, f10b),
                                    (i11a, f11a), (i11b, f11b))]
                for cp in cps:
                    cp.wait()

                def interp_body(j, carry, l=l):
                    s = j * 16
                    a00 = f00a[pl.ds(s, 16)]
                    b00 = f00b[pl.ds(s, 16)]
                    a01 = f01a[pl.ds(s, 16)]
                    b01 = f01b[pl.ds(s, 16)]
                    a10 = f10a[pl.ds(s, 16)]
                    b10 = f10b[pl.ds(s, 16)]
                    a11 = f11a[pl.ds(s, 16)]
                    b11 = f11b[pl.ds(s, 16)]
                    rw = wx_v[pl.ds(s, 16)]
                    cw = wy_v[pl.ds(s, 16)]
                    nrw = 1.0 - rw
                    ncw = 1.0 - cw
                    w00 = ncw * nrw
                    w01 = cw * nrw
                    w10 = ncw * rw
                    w11 = cw * rw
                    o0 = a00 * w00 + a01 * w01 + a10 * w10 + a11 * w11
                    o1 = b00 * w00 + b01 * w01 + b10 * w10 + b11 * w11
                    pos = ((s + iota) << 5) + (2 * l)
                    plsc.store_scatter(out_v, [pos], o0)
                    plsc.store_scatter(out_v, [pos + 1], o1)
                    return carry

                lax.fori_loop(jnp.int32(0), jnp.int32(chunk // 16), interp_body,
                              jnp.int32(0))
                return carry1

            lax.fori_loop(jnp.int32(0), jnp.int32(N_LEVELS), level_body,
                          jnp.int32(0))
            pltpu.sync_copy(out_v, out.at[pl.ds(base * 32, chunk * 32)])
            return carry0

        lax.fori_loop(jnp.int32(0), jnp.int32(n_chunks), chunk_body,
                      jnp.int32(0))

    return ngp_kernel


def kernel(input_coords, tables):
    n = input_coords.shape[0]
    b = jnp.exp((jnp.log(jnp.float32(N_MAX_F)) - jnp.log(jnp.float32(N_MIN_F)))
                / (N_LEVELS - 1))
    res = jnp.stack([jnp.floor(jnp.float32(N_MIN_F) * b ** i)
                     for i in range(N_LEVELS)]).astype(jnp.int32)
    xs = input_coords[:, 0].astype(jnp.int32)
    ys = input_coords[:, 1].astype(jnp.int32)
    tab = tables.reshape(N_LEVELS * T * 2)
    chunk = 2048 if n % (NW * 2048) == 0 else 16
    fn = _make_kernel(n, chunk)
    return fn(tab, xs, ys, res).reshape(n, 32)
